# baseline (speedup 1.0000x reference)
import functools

import jax
import jax.numpy as jnp
from jax import lax
from jax.experimental import pallas as pl
from jax.experimental.pallas import tpu as pltpu
from jax.experimental.pallas import tpu_sc as plsc

_NC = 2
_NS = 16
_NW = _NC * _NS
_L = 16


def _proj_body(x_ref, wt_ref, b_ref, h_ref):
    h_ref[...] = (
        jnp.dot(x_ref[...], wt_ref[...], preferred_element_type=jnp.float32)
        + b_ref[...]
    )


def _project(x, wt, b2):
    n, d = x.shape
    blk = 2000
    return pl.pallas_call(
        _proj_body,
        grid=(n // blk,),
        in_specs=[
            pl.BlockSpec((blk, d), lambda i: (i, 0)),
            pl.BlockSpec((d, d), lambda i: (0, 0)),
            pl.BlockSpec((1, d), lambda i: (0, 0)),
        ],
        out_specs=pl.BlockSpec((blk, d), lambda i: (i, 0)),
        out_shape=jax.ShapeDtypeStruct((n, d), jnp.float32),
    )(x, wt, b2)


@functools.lru_cache(maxsize=None)
def _make_sc(n_nodes, e_total, d, c):
    epw = e_total // _NW
    g_per_c = c // _L
    nchunks = epw // c
    slab_step = (n_nodes // _NS) // 8 * 8
    slab_rows = n_nodes - slab_step * (_NS - 1)
    mesh = plsc.VectorSubcoreMesh(core_axis_name="c", subcore_axis_name="s")

    @functools.partial(
        pl.kernel,
        mesh=mesh,
        compiler_params=pltpu.CompilerParams(needs_layout_passes=False),
        out_type=[
            jax.ShapeDtypeStruct((e_total,), jnp.float32),
            jax.ShapeDtypeStruct((e_total, d), jnp.float32),
        ],
        scratch_types=[
            pltpu.VMEM((c,), jnp.int32),
            pltpu.VMEM((c,), jnp.int32),
            pltpu.VMEM((c, d), jnp.float32),
            pltpu.VMEM((c, d), jnp.float32),
            pltpu.VMEM((c,), jnp.float32),
            pltpu.VMEM((_L * (_L + 1),), jnp.float32),
            pltpu.VMEM_SHARED((n_nodes, d), jnp.float32),
            pltpu.SemaphoreType.DMA,
            pltpu.SemaphoreType.DMA,
        ],
    )
    def sc_kern(h_hbm, src_hbm, dst_hbm, score_out, hdst_out,
                sidx, didx, srows, drows, scv, part, h_sp, sem1, sem2):
        sid = lax.axis_index("s")
        wid = sid * _NC + lax.axis_index("c")
        base = wid * epw

        pltpu.sync_copy(h_hbm.at[pl.ds(sid * slab_step, slab_rows)],
                        h_sp.at[pl.ds(sid * slab_step, slab_rows)])
        plsc.subcore_barrier()

        def chunk_body(ci, carry):
            cbase = base + ci * c
            pltpu.sync_copy(src_hbm.at[pl.ds(cbase, c)], sidx)
            pltpu.sync_copy(dst_hbm.at[pl.ds(cbase, c)], didx)
            cp1 = pltpu.async_copy(h_sp.at[sidx], srows, sem1)
            cp2 = pltpu.async_copy(h_sp.at[didx], drows, sem2)
            cp1.wait()
            cp2.wait()

            lane = lax.iota(jnp.int32, 16)

            def group_body(g, carry2):
                e0 = g * _L
                for e in range(_L):
                    acc = (srows[e0 + e, pl.ds(0, 16)]
                           * drows[e0 + e, pl.ds(0, 16)])
                    for j in range(1, d // 16):
                        acc = acc + (srows[e0 + e, pl.ds(j * 16, 16)]
                                     * drows[e0 + e, pl.ds(j * 16, 16)])
                    part[pl.ds(e * (_L + 1), 16)] = acc
                tot = jnp.zeros((16,), jnp.float32)
                lane17 = lane * (_L + 1)
                for k in range(16):
                    tot = tot + plsc.load_gather(part, [lane17 + k])
                scv[pl.ds(e0, 16)] = 1.0 / (1.0 + jnp.exp(-tot))
                return carry2

            lax.fori_loop(0, g_per_c, group_body, 0)
            pltpu.sync_copy(drows, hdst_out.at[pl.ds(cbase, c)])
            pltpu.sync_copy(scv, score_out.at[pl.ds(cbase, c)])
            return carry

        lax.fori_loop(0, nchunks, chunk_body, 0)

    return sc_kern


def kernel(x, edge_index, W, b):
    e_total = edge_index.shape[1]
    d = x.shape[1]
    src = edge_index[0]
    dst = edge_index[1]
    h = _project(x, W.T, b.reshape(1, d))
    score, h_dst = _make_sc(x.shape[0], e_total, d, 80)(h, src, dst)
    return score.reshape(e_total, 1), h_dst, src, dst


# speedup vs baseline: 1.6169x; 1.6169x over previous
import functools

import jax
import jax.numpy as jnp
from jax import lax
from jax.experimental import pallas as pl
from jax.experimental.pallas import tpu as pltpu
from jax.experimental.pallas import tpu_sc as plsc

_NC = 2
_NS = 16
_NW = _NC * _NS
_L = 16


def _proj_body(x_ref, wt_ref, b_ref, h_ref):
    h_ref[...] = (
        jnp.dot(x_ref[...], wt_ref[...], preferred_element_type=jnp.float32)
        + b_ref[...]
    )


def _project(x, wt, b2):
    n, d = x.shape
    blk = 2000
    return pl.pallas_call(
        _proj_body,
        grid=(n // blk,),
        in_specs=[
            pl.BlockSpec((blk, d), lambda i: (i, 0)),
            pl.BlockSpec((d, d), lambda i: (0, 0)),
            pl.BlockSpec((1, d), lambda i: (0, 0)),
        ],
        out_specs=pl.BlockSpec((blk, d), lambda i: (i, 0)),
        out_shape=jax.ShapeDtypeStruct((n, d), jnp.float32),
    )(x, wt, b2)


@functools.lru_cache(maxsize=None)
def _make_sc(n_nodes, e_total, d, c):
    epw = e_total // _NW
    g_full = c // _L
    tail = c % _L
    nchunks = epw // c
    assert nchunks % 2 == 0 and c % 8 == 0
    mesh = plsc.VectorSubcoreMesh(core_axis_name="c", subcore_axis_name="s")

    @functools.partial(
        pl.kernel,
        mesh=mesh,
        compiler_params=pltpu.CompilerParams(needs_layout_passes=False),
        out_type=[
            jax.ShapeDtypeStruct((e_total,), jnp.float32),
            jax.ShapeDtypeStruct((e_total, d), jnp.float32),
        ],
        scratch_types=[
            [pltpu.VMEM((c,), jnp.int32) for _ in range(2)],
            [pltpu.VMEM((c,), jnp.int32) for _ in range(2)],
            [pltpu.VMEM((c + _L, d), jnp.float32) for _ in range(2)],
            [pltpu.VMEM((c + _L, d), jnp.float32) for _ in range(2)],
            pltpu.VMEM((epw + _L,), jnp.float32),
            pltpu.VMEM((_L * (_L + 1),), jnp.float32),
            [pltpu.SemaphoreType.DMA for _ in range(2)],
            [pltpu.SemaphoreType.DMA for _ in range(2)],
        ],
    )
    def sc_kern(h_hbm, src_hbm, dst_hbm, score_out, hdst_out,
                sidx, didx, srows, drows, scv, part, gsem, wsem):
        wid = lax.axis_index("s") * _NC + lax.axis_index("c")
        base = wid * epw
        lane = lax.iota(jnp.int32, 16)

        def issue_gathers(ci, p):
            cbase = base + ci * c
            pltpu.sync_copy(src_hbm.at[pl.ds(cbase, c)], sidx[p])
            pltpu.sync_copy(dst_hbm.at[pl.ds(cbase, c)], didx[p])
            pltpu.async_copy(h_hbm.at[sidx[p]], srows[p].at[pl.ds(0, c)],
                             gsem[p])
            pltpu.async_copy(h_hbm.at[didx[p]], drows[p].at[pl.ds(0, c)],
                             gsem[p])

        def wait_gathers(p):
            pltpu.make_async_copy(h_hbm.at[sidx[p]], srows[p].at[pl.ds(0, c)],
                                  gsem[p]).wait()
            pltpu.make_async_copy(h_hbm.at[didx[p]], drows[p].at[pl.ds(0, c)],
                                  gsem[p]).wait()

        def compute_chunk(ci, p):
            sr, dr = srows[p], drows[p]

            def group(g):
                e0 = g * _L
                for e in range(_L):
                    acc = (sr[e0 + e, pl.ds(0, 16)]
                           * dr[e0 + e, pl.ds(0, 16)])
                    for j in range(1, d // 16):
                        acc = acc + (sr[e0 + e, pl.ds(j * 16, 16)]
                                     * dr[e0 + e, pl.ds(j * 16, 16)])
                    part[pl.ds(e * (_L + 1), 16)] = acc
                tot = jnp.zeros((16,), jnp.float32)
                lane17 = lane * (_L + 1)
                for k in range(16):
                    tot = tot + plsc.load_gather(part, [lane17 + k])
                scv[pl.ds(ci * c + e0, 16)] = 1.0 / (1.0 + jnp.exp(-tot))

            def group_body(g, carry2):
                group(g)
                return carry2

            lax.fori_loop(0, g_full, group_body, 0)
            if tail:
                group(g_full)

        for p in range(2):
            issue_gathers(p, p)

        def outer(t, carry):
            for p in range(2):
                ci = 2 * t + p
                wait_gathers(p)
                pltpu.async_copy(drows[p].at[pl.ds(0, c)],
                                 hdst_out.at[pl.ds(base + ci * c, c)], wsem[p])
                compute_chunk(ci, p)

                @pl.when(ci + 2 < nchunks)
                def _prefetch():
                    pltpu.make_async_copy(
                        drows[p].at[pl.ds(0, c)],
                        hdst_out.at[pl.ds(base, c)], wsem[p]).wait()
                    issue_gathers(ci + 2, p)
            return carry

        lax.fori_loop(0, nchunks // 2, outer, 0)
        for p in range(2):
            pltpu.make_async_copy(
                drows[p].at[pl.ds(0, c)],
                hdst_out.at[pl.ds(base, c)], wsem[p]).wait()
        pltpu.sync_copy(scv.at[pl.ds(0, epw)], score_out.at[pl.ds(base, epw)])

    return sc_kern


def kernel(x, edge_index, W, b):
    e_total = edge_index.shape[1]
    d = x.shape[1]
    src = edge_index[0]
    dst = edge_index[1]
    h = _project(x, W.T, b.reshape(1, d))
    score, h_dst = _make_sc(x.shape[0], e_total, d, 200)(h, src, dst)
    return score.reshape(e_total, 1), h_dst, src, dst
